# baseline (speedup 1.0000x reference)
import functools

import jax
import jax.numpy as jnp
from jax import lax
from jax.experimental import pallas as pl
from jax.experimental.pallas import tpu as pltpu
from jax.experimental.pallas import tpu_sc as plsc

N, E, D, NB, M = 10000, 320000, 128, 8, 16
RMAX = 5.0
AVG_NEIGH = 32.0

NCORES, NSUB = 2, 16
NW = NCORES * NSUB
C = 80
EPW = E // NW
NCH = EPW // C
N_PAD = 10240
ROWS_PT = N_PAD // NSUB

_mesh = plsc.VectorSubcoreMesh(core_axis_name="c", subcore_axis_name="s")


@functools.partial(
    pl.kernel,
    out_type=jax.ShapeDtypeStruct((NW, NCH, C, 16), jnp.float32),
    mesh=_mesh,
    scratch_types=[
        pltpu.VMEM((2, 2, C), jnp.int32),
        pltpu.VMEM((C, D), jnp.float32),
        pltpu.VMEM((C, D), jnp.float32),
        pltpu.VMEM((C, D), jnp.float32),
        pltpu.VMEM((C, D), jnp.float32),
        pltpu.VMEM((C, 16), jnp.float32),
        pltpu.VMEM((C, 16), jnp.float32),
        pltpu.SemaphoreType.DMA,
        pltpu.SemaphoreType.DMA,
        pltpu.SemaphoreType.DMA,
        pltpu.SemaphoreType.DMA,
    ],
)
def _edge_vec_sc(pos_hbm, ei_hbm, out_hbm,
                 idx_v, a_v0, a_v1, b_v0, b_v1, o_v0, o_v1,
                 sem_d0, sem_d1, sem_o0, sem_o1):
    cid = lax.axis_index("c")
    sid = lax.axis_index("s")
    wid = cid * NSUB + sid
    a_bufs = (a_v0, a_v1)
    b_bufs = (b_v0, b_v1)
    o_bufs = (o_v0, o_v1)
    sem_d = (sem_d0, sem_d1)
    sem_o = (sem_o0, sem_o1)

    def issue_dat(b):
        pltpu.async_copy(pos_hbm.at[idx_v.at[b, 0]], a_bufs[b], sem_d[b])
        pltpu.async_copy(pos_hbm.at[idx_v.at[b, 1]], b_bufs[b], sem_d[b])

    def wait_dat(b):
        pltpu.make_async_copy(pos_hbm.at[idx_v.at[b, 0]], a_bufs[b],
                              sem_d[b]).wait()
        pltpu.make_async_copy(pos_hbm.at[idx_v.at[b, 1]], b_bufs[b],
                              sem_d[b]).wait()

    def compute(b):
        av, bv, ov = a_bufs[b], b_bufs[b], o_bufs[b]

        def row(i, c2):
            sl = pl.ds(0, 16)
            ov[i, :] = bv[i, sl] - av[i, sl]
            return c2

        lax.fori_loop(0, C, row, 0)

    pltpu.sync_copy(ei_hbm.at[wid, pl.ds(0, 2)], idx_v)
    issue_dat(0)
    issue_dat(1)

    def pair(p, carry):
        wait_dat(0)
        compute(0)
        wait_dat(1)
        compute(1)
        c0 = pltpu.async_copy(o_v0, out_hbm.at[wid, 2 * p], sem_o0)
        c1 = pltpu.async_copy(o_v1, out_hbm.at[wid, 2 * p + 1], sem_o1)
        c0.wait()
        c1.wait()

        @pl.when(p < (NCH - 1) // 2 - 1)
        def _():
            pltpu.sync_copy(ei_hbm.at[wid, pl.ds(2 * p + 2, 2)], idx_v)
            issue_dat(0)
            issue_dat(1)
        return carry

    lax.fori_loop(0, (NCH - 1) // 2, pair, 0)
    pltpu.sync_copy(ei_hbm.at[wid, pl.ds(NCH - 1, 1)],
                    idx_v.at[pl.ds(0, 1)])
    issue_dat(0)
    wait_dat(0)
    compute(0)
    pltpu.sync_copy(o_v0, out_hbm.at[wid, NCH - 1])


@functools.partial(
    pl.kernel,
    out_type=jax.ShapeDtypeStruct((NCORES, NSUB, ROWS_PT, D), jnp.float32),
    mesh=_mesh,
    scratch_types=[
        pltpu.VMEM((2, 2, C), jnp.int32),
        pltpu.VMEM((C, D), jnp.float32),
        pltpu.VMEM((C, D), jnp.float32),
        pltpu.VMEM((C, D), jnp.float32),
        pltpu.VMEM((C, D), jnp.float32),
        pltpu.VMEM_SHARED((N_PAD, D), jnp.float32),
        pltpu.SemaphoreType.DMA,
        pltpu.SemaphoreType.DMA,
        pltpu.SemaphoreType.DMA,
        pltpu.SemaphoreType.DMA,
    ],
)
def _interact_sc(h_hbm, w_hbm, ei_hbm, out_hbm,
                 idx_v, w_v0, w_v1, r_v0, r_v1, agg_sh,
                 sem_d0, sem_d1, sem_s0, sem_s1):
    cid = lax.axis_index("c")
    sid = lax.axis_index("s")
    wid = cid * NSUB + sid
    base = pl.multiple_of(sid * ROWS_PT, 8)
    w_bufs = (w_v0, w_v1)
    r_bufs = (r_v0, r_v1)
    sem_d = (sem_d0, sem_d1)
    sem_s = (sem_s0, sem_s1)

    def zrow(i, c2):
        for k in range(D // 16):
            r_v0[i, pl.ds(k * 16, 16)] = jnp.zeros((16,), jnp.float32)
        return c2

    lax.fori_loop(0, C, zrow, 0)
    for t in range(ROWS_PT // C):
        pltpu.sync_copy(r_v0, agg_sh.at[pl.ds(base + t * C, C)])
    plsc.subcore_barrier()

    def issue_dat(b, c):
        pltpu.async_copy(h_hbm.at[idx_v.at[b, 0]], r_bufs[b], sem_d[b])
        pltpu.async_copy(w_hbm.at[wid, c], w_bufs[b], sem_d[b])

    def wait_dat(b):
        pltpu.make_async_copy(h_hbm.at[idx_v.at[b, 0]], r_bufs[b],
                              sem_d[b]).wait()
        pltpu.make_async_copy(w_hbm.at[wid, 0], w_bufs[b], sem_d[b]).wait()

    def mul(b):
        rv, wv = r_bufs[b], w_bufs[b]

        def row(i, c2):
            for k in range(D // 16):
                sl = pl.ds(k * 16, 16)
                rv[i, sl] = rv[i, sl] * wv[i, sl]
            return c2

        lax.fori_loop(0, C, row, 0)

    pltpu.sync_copy(ei_hbm.at[wid, pl.ds(0, 2)], idx_v)
    issue_dat(0, 0)
    issue_dat(1, 1)

    def pair(p, carry):
        wait_dat(0)
        mul(0)
        wait_dat(1)
        mul(1)

        c0 = pltpu.async_copy(r_v0, agg_sh.at[idx_v.at[0, 1]], sem_s0,
                              add=True)
        c1 = pltpu.async_copy(r_v1, agg_sh.at[idx_v.at[1, 1]], sem_s1,
                              add=True)
        c0.wait()
        c1.wait()

        @pl.when(p < (NCH - 1) // 2 - 1)
        def _():
            pltpu.sync_copy(ei_hbm.at[wid, pl.ds(2 * p + 2, 2)], idx_v)
            issue_dat(0, 2 * p + 2)
            issue_dat(1, 2 * p + 3)
        return carry

    lax.fori_loop(0, (NCH - 1) // 2, pair, 0)
    pltpu.sync_copy(ei_hbm.at[wid, pl.ds(NCH - 1, 1)],
                    idx_v.at[pl.ds(0, 1)])
    issue_dat(0, NCH - 1)
    wait_dat(0)
    mul(0)
    pltpu.sync_copy(r_v0, agg_sh.at[idx_v.at[0, 1]], add=True)
    plsc.subcore_barrier()
    pltpu.sync_copy(agg_sh.at[pl.ds(base, ROWS_PT)], out_hbm.at[cid, sid])


@functools.partial(
    pl.kernel,
    out_type=jax.ShapeDtypeStruct((NW, NCH, C, M), jnp.float32),
    mesh=_mesh,
    scratch_types=[
        pltpu.VMEM((2, 2, C), jnp.int32),
        pltpu.VMEM((C, D), jnp.float32),
        pltpu.VMEM((C, D), jnp.float32),
        pltpu.VMEM((C, D), jnp.float32),
        pltpu.VMEM((C, D), jnp.float32),
        pltpu.VMEM((C, M), jnp.float32),
        pltpu.VMEM((C, M), jnp.float32),
        pltpu.SemaphoreType.DMA,
        pltpu.SemaphoreType.DMA,
        pltpu.SemaphoreType.DMA,
        pltpu.SemaphoreType.DMA,
    ],
)
def _readout_sc(eh_hbm, gate_hbm, ei_hbm, out_hbm,
                idx_v, a_v0, a_v1, b_v0, b_v1, g_v0, g_v1,
                sem_d0, sem_d1, sem_o0, sem_o1):
    cid = lax.axis_index("c")
    sid = lax.axis_index("s")
    wid = cid * NSUB + sid
    a_bufs = (a_v0, a_v1)
    b_bufs = (b_v0, b_v1)
    g_bufs = (g_v0, g_v1)
    sem_d = (sem_d0, sem_d1)
    sem_o = (sem_o0, sem_o1)

    def issue_dat(b, c):
        pltpu.async_copy(eh_hbm.at[idx_v.at[b, 0]], a_bufs[b], sem_d[b])
        pltpu.async_copy(eh_hbm.at[idx_v.at[b, 1]], b_bufs[b], sem_d[b])
        pltpu.async_copy(gate_hbm.at[wid, c], g_bufs[b], sem_d[b])

    def wait_dat(b):
        pltpu.make_async_copy(eh_hbm.at[idx_v.at[b, 0]], a_bufs[b],
                              sem_d[b]).wait()
        pltpu.make_async_copy(eh_hbm.at[idx_v.at[b, 1]], b_bufs[b],
                              sem_d[b]).wait()
        pltpu.make_async_copy(gate_hbm.at[wid, 0], g_bufs[b],
                              sem_d[b]).wait()

    def compute(b):
        av, bv, gv = a_bufs[b], b_bufs[b], g_bufs[b]

        def row(i, c2):
            sl = pl.ds(0, M)
            gv[i, :] = (av[i, sl] + bv[i, sl]) * gv[i, :]
            return c2

        lax.fori_loop(0, C, row, 0)

    pltpu.sync_copy(ei_hbm.at[wid, pl.ds(0, 2)], idx_v)
    issue_dat(0, 0)
    issue_dat(1, 1)

    def pair(p, carry):
        wait_dat(0)
        compute(0)
        wait_dat(1)
        compute(1)
        c0 = pltpu.async_copy(g_v0, out_hbm.at[wid, 2 * p], sem_o0)
        c1 = pltpu.async_copy(g_v1, out_hbm.at[wid, 2 * p + 1], sem_o1)
        c0.wait()
        c1.wait()

        @pl.when(p < (NCH - 1) // 2 - 1)
        def _():
            pltpu.sync_copy(ei_hbm.at[wid, pl.ds(2 * p + 2, 2)], idx_v)
            issue_dat(0, 2 * p + 2)
            issue_dat(1, 2 * p + 3)
        return carry

    lax.fori_loop(0, (NCH - 1) // 2, pair, 0)
    pltpu.sync_copy(ei_hbm.at[wid, pl.ds(NCH - 1, 1)],
                    idx_v.at[pl.ds(0, 1)])
    issue_dat(0, NCH - 1)
    wait_dat(0)
    compute(0)
    pltpu.sync_copy(g_v0, out_hbm.at[wid, NCH - 1])


def _embed_body(na_ref, we_ref, out_ref):
    out_ref[...] = jnp.dot(na_ref[...], we_ref[...],
                           preferred_element_type=jnp.float32)


def _silu(x):
    return x * jax.nn.sigmoid(x)


BE = 2000


def _edge_dense_body(vec_ref, sh_ref, w1cat, w2blk, wsh1, wsh2,
                     w1_ref, w2_ref, gate_ref):
    v = vec_ref[...]
    v3 = v[:, 0:3] + sh_ref[...]
    r2 = jnp.sum(v3 * v3, axis=1, keepdims=True) + 1e-12
    r = jnp.sqrt(r2)
    inv_r = 1.0 / r
    ux = v3[:, 0:1] * inv_r
    uy = v3[:, 1:2] * inv_r
    uz = v3[:, 2:3] * inv_r

    theta = (jnp.pi / RMAX) * r
    s1 = jnp.sin(theta)
    c2 = 2.0 * jnp.cos(theta)
    sins = [s1, c2 * s1]
    for _ in range(NB - 2):
        sins.append(c2 * sins[-1] - sins[-2])
    bessel = jnp.sqrt(2.0 / RMAX) * jnp.concatenate(sins, axis=1) * inv_r
    u = jnp.clip(r / RMAX, 0.0, 1.0)
    p = 6.0
    fc = (1.0 - ((p + 1.0) * (p + 2.0) / 2.0) * u ** 6
          + p * (p + 2.0) * u ** 7 - (p * (p + 1.0) / 2.0) * u ** 8)
    ef = bessel * fc

    c0 = 0.28209479177
    c1 = 0.4886025119

    z = _silu(jnp.dot(ef, w1cat[...], preferred_element_type=jnp.float32))
    y = jnp.dot(z, w2blk[...], preferred_element_type=jnp.float32)

    def shw(w_sh):
        return (c0 * w_sh[0:1, :]
                + c1 * (ux * w_sh[1:2, :] + uy * w_sh[2:3, :]
                        + uz * w_sh[3:4, :]))

    w1_ref[...] = y[:, 0:D] * shw(wsh1)
    w2_ref[...] = y[:, D:2 * D] * shw(wsh2)
    gate_ref[...] = y[:, 2 * D:2 * D + M]


def _merge_agg(m_ref):
    return m_ref[0, :N, :] + m_ref[1, :N, :]


def _node1_body(m_ref, wl_ref, h1_ref):
    agg = _merge_agg(m_ref)
    h1_ref[...] = _silu(jnp.dot(agg, wl_ref[...],
                                preferred_element_type=jnp.float32)) / AVG_NEIGH


def _node2_body(m_ref, h1_ref, wl_ref, wn1, wn2, we1, we2,
                nl_ref, eh_ref):
    agg = _merge_agg(m_ref)
    h1 = h1_ref[...]
    h2 = _silu(jnp.dot(agg, wl_ref[...],
                       preferred_element_type=jnp.float32)) / AVG_NEIGH
    nl_ref[...] = (jnp.dot(h1, wn1[...], preferred_element_type=jnp.float32)
                   + jnp.dot(h2, wn2[...], preferred_element_type=jnp.float32))
    eh = (jnp.dot(h1, we1[...], preferred_element_type=jnp.float32)
          + jnp.dot(h2, we2[...], preferred_element_type=jnp.float32))
    eh_ref[...] = jnp.pad(eh, ((0, 0), (0, D - M)))


def _full(shape):
    return pl.BlockSpec(shape, lambda i: tuple(0 for _ in shape))


def kernel(positions, edge_index, shifts, node_attrs, W_embed,
           W_r1a, W_r2a, W_sh1, W_lin1, W_r1b, W_r2b, W_sh2, W_lin2,
           W_node, W_eh, W_er1, W_er2):
    ei4 = edge_index.reshape(2, NW, NCH, C).transpose(1, 2, 0, 3)
    pos128 = jnp.pad(positions, ((0, 0), (0, D - 3)))

    h0 = pl.pallas_call(
        _embed_body,
        out_shape=jax.ShapeDtypeStruct((N, D), jnp.float32),
    )(node_attrs, W_embed)

    vec = _edge_vec_sc(pos128, ei4)

    w1cat = jnp.concatenate([W_r1a, W_r1b, W_er1], axis=1)
    w2blk = jnp.zeros((192, 2 * D + M), jnp.float32)
    w2blk = w2blk.at[0:64, 0:D].set(W_r2a)
    w2blk = w2blk.at[64:128, D:2 * D].set(W_r2b)
    w2blk = w2blk.at[128:192, 2 * D:2 * D + M].set(W_er2)

    w1, w2, gate = pl.pallas_call(
        _edge_dense_body,
        grid=(E // BE,),
        in_specs=[
            pl.BlockSpec((BE, 16), lambda i: (i, 0)),
            pl.BlockSpec((BE, 3), lambda i: (i, 0)),
            _full((NB, 192)), _full((192, 2 * D + M)),
            _full((4, D)), _full((4, D)),
        ],
        out_specs=[
            pl.BlockSpec((BE, D), lambda i: (i, 0)),
            pl.BlockSpec((BE, D), lambda i: (i, 0)),
            pl.BlockSpec((BE, M), lambda i: (i, 0)),
        ],
        out_shape=[
            jax.ShapeDtypeStruct((E, D), jnp.float32),
            jax.ShapeDtypeStruct((E, D), jnp.float32),
            jax.ShapeDtypeStruct((E, M), jnp.float32),
        ],
    )(vec.reshape(E, 16), shifts, w1cat, w2blk, W_sh1, W_sh2)

    m1 = _interact_sc(h0, w1.reshape(NW, NCH, C, D), ei4)

    h1 = pl.pallas_call(
        _node1_body,
        out_shape=jax.ShapeDtypeStruct((N, D), jnp.float32),
    )(m1.reshape(NCORES, N_PAD, D), W_lin1)

    m2 = _interact_sc(h1, w2.reshape(NW, NCH, C, D), ei4)

    node_labels, eh = pl.pallas_call(
        _node2_body,
        out_shape=[
            jax.ShapeDtypeStruct((N, M), jnp.float32),
            jax.ShapeDtypeStruct((N, D), jnp.float32),
        ],
    )(m2.reshape(NCORES, N_PAD, D), h1, W_lin2,
      W_node[:D], W_node[D:], W_eh[:D], W_eh[D:])

    edge_labels = _readout_sc(eh, gate.reshape(NW, NCH, C, M), ei4)
    return node_labels, edge_labels.reshape(E, M)


# speedup vs baseline: 1.1258x; 1.1258x over previous
import functools

import jax
import jax.numpy as jnp
from jax import lax
from jax.experimental import pallas as pl
from jax.experimental.pallas import tpu as pltpu
from jax.experimental.pallas import tpu_sc as plsc

N, E, D, NB, M = 10000, 320000, 128, 8, 16
RMAX = 5.0
AVG_NEIGH = 32.0

NCORES, NSUB = 2, 16
NW = NCORES * NSUB
C = 80
EPW = E // NW
NCH = EPW // C
N_PAD = 10240
ROWS_PT = N_PAD // NSUB

_mesh = plsc.VectorSubcoreMesh(core_axis_name="c", subcore_axis_name="s")


@functools.partial(
    pl.kernel,
    out_type=jax.ShapeDtypeStruct((NW, NCH, C, 16), jnp.float32),
    mesh=_mesh,
    compiler_params=pltpu.CompilerParams(use_tc_tiling_on_sc=False),
    scratch_types=[
        pltpu.VMEM((2, 2, C), jnp.int32),
        pltpu.VMEM((C, 16), jnp.float32),
        pltpu.VMEM((C, 16), jnp.float32),
        pltpu.VMEM((C, 16), jnp.float32),
        pltpu.VMEM((C, 16), jnp.float32),
        pltpu.VMEM((C, 16), jnp.float32),
        pltpu.VMEM((C, 16), jnp.float32),
        pltpu.SemaphoreType.DMA,
        pltpu.SemaphoreType.DMA,
        pltpu.SemaphoreType.DMA,
        pltpu.SemaphoreType.DMA,
    ],
)
def _edge_vec_sc(pos_hbm, ei_hbm, out_hbm,
                 idx_v, a_v0, a_v1, b_v0, b_v1, o_v0, o_v1,
                 sem_d0, sem_d1, sem_o0, sem_o1):
    cid = lax.axis_index("c")
    sid = lax.axis_index("s")
    wid = cid * NSUB + sid
    a_bufs = (a_v0, a_v1)
    b_bufs = (b_v0, b_v1)
    o_bufs = (o_v0, o_v1)
    sem_d = (sem_d0, sem_d1)
    sem_o = (sem_o0, sem_o1)

    def issue_dat(b):
        pltpu.async_copy(pos_hbm.at[idx_v.at[b, 0]], a_bufs[b], sem_d[b])
        pltpu.async_copy(pos_hbm.at[idx_v.at[b, 1]], b_bufs[b], sem_d[b])

    def wait_dat(b):
        pltpu.make_async_copy(pos_hbm.at[idx_v.at[b, 0]], a_bufs[b],
                              sem_d[b]).wait()
        pltpu.make_async_copy(pos_hbm.at[idx_v.at[b, 1]], b_bufs[b],
                              sem_d[b]).wait()

    def compute(b):
        av, bv, ov = a_bufs[b], b_bufs[b], o_bufs[b]

        def row(i, c2):
            ov[i, :] = bv[i, :] - av[i, :]
            return c2

        lax.fori_loop(0, C, row, 0)

    pltpu.sync_copy(ei_hbm.at[wid, pl.ds(0, 2)], idx_v)
    issue_dat(0)
    issue_dat(1)

    def pair(p, carry):
        wait_dat(0)
        compute(0)
        wait_dat(1)
        compute(1)
        c0 = pltpu.async_copy(o_v0, out_hbm.at[wid, 2 * p], sem_o0)
        c1 = pltpu.async_copy(o_v1, out_hbm.at[wid, 2 * p + 1], sem_o1)
        c0.wait()
        c1.wait()

        @pl.when(p < (NCH - 1) // 2 - 1)
        def _():
            pltpu.sync_copy(ei_hbm.at[wid, pl.ds(2 * p + 2, 2)], idx_v)
            issue_dat(0)
            issue_dat(1)
        return carry

    lax.fori_loop(0, (NCH - 1) // 2, pair, 0)
    pltpu.sync_copy(ei_hbm.at[wid, pl.ds(NCH - 1, 1)],
                    idx_v.at[pl.ds(0, 1)])
    issue_dat(0)
    wait_dat(0)
    compute(0)
    pltpu.sync_copy(o_v0, out_hbm.at[wid, NCH - 1])


@functools.partial(
    pl.kernel,
    out_type=jax.ShapeDtypeStruct((NCORES, NSUB, ROWS_PT, D), jnp.float32),
    mesh=_mesh,
    scratch_types=[
        pltpu.VMEM((2, 2, C), jnp.int32),
        pltpu.VMEM((C, D), jnp.float32),
        pltpu.VMEM((C, D), jnp.float32),
        pltpu.VMEM((C, D), jnp.float32),
        pltpu.VMEM((C, D), jnp.float32),
        pltpu.VMEM_SHARED((N_PAD, D), jnp.float32),
        pltpu.SemaphoreType.DMA,
        pltpu.SemaphoreType.DMA,
        pltpu.SemaphoreType.DMA,
        pltpu.SemaphoreType.DMA,
    ],
)
def _interact_sc(h_hbm, w_hbm, ei_hbm, out_hbm,
                 idx_v, w_v0, w_v1, r_v0, r_v1, agg_sh,
                 sem_d0, sem_d1, sem_s0, sem_s1):
    cid = lax.axis_index("c")
    sid = lax.axis_index("s")
    wid = cid * NSUB + sid
    base = pl.multiple_of(sid * ROWS_PT, 8)
    w_bufs = (w_v0, w_v1)
    r_bufs = (r_v0, r_v1)
    sem_d = (sem_d0, sem_d1)
    sem_s = (sem_s0, sem_s1)

    def zrow(i, c2):
        for k in range(D // 16):
            r_v0[i, pl.ds(k * 16, 16)] = jnp.zeros((16,), jnp.float32)
        return c2

    lax.fori_loop(0, C, zrow, 0)
    for t in range(ROWS_PT // C):
        pltpu.sync_copy(r_v0, agg_sh.at[pl.ds(base + t * C, C)])
    plsc.subcore_barrier()

    def issue_dat(b, c):
        pltpu.async_copy(h_hbm.at[idx_v.at[b, 0]], r_bufs[b], sem_d[b])
        pltpu.async_copy(w_hbm.at[wid, c], w_bufs[b], sem_d[b])

    def wait_dat(b):
        pltpu.make_async_copy(h_hbm.at[idx_v.at[b, 0]], r_bufs[b],
                              sem_d[b]).wait()
        pltpu.make_async_copy(w_hbm.at[wid, 0], w_bufs[b], sem_d[b]).wait()

    def mul(b):
        rv, wv = r_bufs[b], w_bufs[b]

        def row(i, c2):
            for k in range(D // 16):
                sl = pl.ds(k * 16, 16)
                rv[i, sl] = rv[i, sl] * wv[i, sl]
            return c2

        lax.fori_loop(0, C, row, 0)

    pltpu.sync_copy(ei_hbm.at[wid, pl.ds(0, 2)], idx_v)
    issue_dat(0, 0)
    issue_dat(1, 1)

    def pair(p, carry):
        wait_dat(0)
        mul(0)
        wait_dat(1)
        mul(1)

        c0 = pltpu.async_copy(r_v0, agg_sh.at[idx_v.at[0, 1]], sem_s0,
                              add=True)
        c1 = pltpu.async_copy(r_v1, agg_sh.at[idx_v.at[1, 1]], sem_s1,
                              add=True)
        c0.wait()
        c1.wait()

        @pl.when(p < (NCH - 1) // 2 - 1)
        def _():
            pltpu.sync_copy(ei_hbm.at[wid, pl.ds(2 * p + 2, 2)], idx_v)
            issue_dat(0, 2 * p + 2)
            issue_dat(1, 2 * p + 3)
        return carry

    lax.fori_loop(0, (NCH - 1) // 2, pair, 0)
    pltpu.sync_copy(ei_hbm.at[wid, pl.ds(NCH - 1, 1)],
                    idx_v.at[pl.ds(0, 1)])
    issue_dat(0, NCH - 1)
    wait_dat(0)
    mul(0)
    pltpu.sync_copy(r_v0, agg_sh.at[idx_v.at[0, 1]], add=True)
    plsc.subcore_barrier()
    pltpu.sync_copy(agg_sh.at[pl.ds(base, ROWS_PT)], out_hbm.at[cid, sid])


@functools.partial(
    pl.kernel,
    out_type=jax.ShapeDtypeStruct((NW, NCH, C, M), jnp.float32),
    mesh=_mesh,
    compiler_params=pltpu.CompilerParams(use_tc_tiling_on_sc=False),
    scratch_types=[
        pltpu.VMEM((2, 2, C), jnp.int32),
        pltpu.VMEM((C, M), jnp.float32),
        pltpu.VMEM((C, M), jnp.float32),
        pltpu.VMEM((C, M), jnp.float32),
        pltpu.VMEM((C, M), jnp.float32),
        pltpu.VMEM((C, M), jnp.float32),
        pltpu.VMEM((C, M), jnp.float32),
        pltpu.SemaphoreType.DMA,
        pltpu.SemaphoreType.DMA,
        pltpu.SemaphoreType.DMA,
        pltpu.SemaphoreType.DMA,
    ],
)
def _readout_sc(eh_hbm, gate_hbm, ei_hbm, out_hbm,
                idx_v, a_v0, a_v1, b_v0, b_v1, g_v0, g_v1,
                sem_d0, sem_d1, sem_o0, sem_o1):
    cid = lax.axis_index("c")
    sid = lax.axis_index("s")
    wid = cid * NSUB + sid
    a_bufs = (a_v0, a_v1)
    b_bufs = (b_v0, b_v1)
    g_bufs = (g_v0, g_v1)
    sem_d = (sem_d0, sem_d1)
    sem_o = (sem_o0, sem_o1)

    def issue_dat(b, c):
        pltpu.async_copy(eh_hbm.at[idx_v.at[b, 0]], a_bufs[b], sem_d[b])
        pltpu.async_copy(eh_hbm.at[idx_v.at[b, 1]], b_bufs[b], sem_d[b])
        pltpu.async_copy(gate_hbm.at[wid, c], g_bufs[b], sem_d[b])

    def wait_dat(b):
        pltpu.make_async_copy(eh_hbm.at[idx_v.at[b, 0]], a_bufs[b],
                              sem_d[b]).wait()
        pltpu.make_async_copy(eh_hbm.at[idx_v.at[b, 1]], b_bufs[b],
                              sem_d[b]).wait()
        pltpu.make_async_copy(gate_hbm.at[wid, 0], g_bufs[b],
                              sem_d[b]).wait()

    def compute(b):
        av, bv, gv = a_bufs[b], b_bufs[b], g_bufs[b]

        def row(i, c2):
            gv[i, :] = (av[i, :] + bv[i, :]) * gv[i, :]
            return c2

        lax.fori_loop(0, C, row, 0)

    pltpu.sync_copy(ei_hbm.at[wid, pl.ds(0, 2)], idx_v)
    issue_dat(0, 0)
    issue_dat(1, 1)

    def pair(p, carry):
        wait_dat(0)
        compute(0)
        wait_dat(1)
        compute(1)
        c0 = pltpu.async_copy(g_v0, out_hbm.at[wid, 2 * p], sem_o0)
        c1 = pltpu.async_copy(g_v1, out_hbm.at[wid, 2 * p + 1], sem_o1)
        c0.wait()
        c1.wait()

        @pl.when(p < (NCH - 1) // 2 - 1)
        def _():
            pltpu.sync_copy(ei_hbm.at[wid, pl.ds(2 * p + 2, 2)], idx_v)
            issue_dat(0, 2 * p + 2)
            issue_dat(1, 2 * p + 3)
        return carry

    lax.fori_loop(0, (NCH - 1) // 2, pair, 0)
    pltpu.sync_copy(ei_hbm.at[wid, pl.ds(NCH - 1, 1)],
                    idx_v.at[pl.ds(0, 1)])
    issue_dat(0, NCH - 1)
    wait_dat(0)
    compute(0)
    pltpu.sync_copy(g_v0, out_hbm.at[wid, NCH - 1])


def _embed_body(na_ref, we_ref, out_ref):
    out_ref[...] = jnp.dot(na_ref[...], we_ref[...],
                           preferred_element_type=jnp.float32)


def _silu(x):
    return x * jax.nn.sigmoid(x)


BE = 2000


def _edge_dense_body(vec_ref, sh_ref, w1cat, w2blk, wsh1, wsh2,
                     w1_ref, w2_ref, gate_ref):
    v = vec_ref[...]
    v3 = v[:, 0:3] + sh_ref[...]
    r2 = jnp.sum(v3 * v3, axis=1, keepdims=True) + 1e-12
    r = jnp.sqrt(r2)
    inv_r = 1.0 / r
    ux = v3[:, 0:1] * inv_r
    uy = v3[:, 1:2] * inv_r
    uz = v3[:, 2:3] * inv_r

    n = lax.broadcasted_iota(jnp.int32, (1, NB), 1).astype(jnp.float32) + 1.0
    bessel = jnp.sqrt(2.0 / RMAX) * jnp.sin(n * (jnp.pi / RMAX) * r) * inv_r
    u = jnp.clip(r / RMAX, 0.0, 1.0)
    p = 6.0
    fc = (1.0 - ((p + 1.0) * (p + 2.0) / 2.0) * u ** 6
          + p * (p + 2.0) * u ** 7 - (p * (p + 1.0) / 2.0) * u ** 8)
    ef = bessel * fc

    c0 = 0.28209479177
    c1 = 0.4886025119

    z = _silu(jnp.dot(ef, w1cat[...], preferred_element_type=jnp.float32))
    y = jnp.dot(z, w2blk[...], preferred_element_type=jnp.float32)

    def shw(w_sh):
        return (c0 * w_sh[0:1, :]
                + c1 * (ux * w_sh[1:2, :] + uy * w_sh[2:3, :]
                        + uz * w_sh[3:4, :]))

    w1_ref[...] = y[:, 0:D] * shw(wsh1)
    w2_ref[...] = y[:, D:2 * D] * shw(wsh2)
    gate_ref[...] = y[:, 2 * D:2 * D + M]


def _merge_agg(m_ref):
    return m_ref[0, :N, :] + m_ref[1, :N, :]


def _node1_body(m_ref, wl_ref, h1_ref):
    agg = _merge_agg(m_ref)
    h1_ref[...] = _silu(jnp.dot(agg, wl_ref[...],
                                preferred_element_type=jnp.float32)) / AVG_NEIGH


def _node2_body(m_ref, h1_ref, wl_ref, wn1, wn2, we1, we2,
                nl_ref, eh_ref):
    agg = _merge_agg(m_ref)
    h1 = h1_ref[...]
    h2 = _silu(jnp.dot(agg, wl_ref[...],
                       preferred_element_type=jnp.float32)) / AVG_NEIGH
    nl_ref[...] = (jnp.dot(h1, wn1[...], preferred_element_type=jnp.float32)
                   + jnp.dot(h2, wn2[...], preferred_element_type=jnp.float32))
    eh_ref[...] = (jnp.dot(h1, we1[...], preferred_element_type=jnp.float32)
                   + jnp.dot(h2, we2[...], preferred_element_type=jnp.float32))


def _full(shape):
    return pl.BlockSpec(shape, lambda i: tuple(0 for _ in shape))


def kernel(positions, edge_index, shifts, node_attrs, W_embed,
           W_r1a, W_r2a, W_sh1, W_lin1, W_r1b, W_r2b, W_sh2, W_lin2,
           W_node, W_eh, W_er1, W_er2):
    ei4 = edge_index.reshape(2, NW, NCH, C).transpose(1, 2, 0, 3)
    pos16 = jnp.pad(positions, ((0, 0), (0, 13)))

    h0 = pl.pallas_call(
        _embed_body,
        out_shape=jax.ShapeDtypeStruct((N, D), jnp.float32),
    )(node_attrs, W_embed)

    vec = _edge_vec_sc(pos16, ei4)

    w1cat = jnp.concatenate([W_r1a, W_r1b, W_er1], axis=1)
    w2blk = jnp.zeros((192, 2 * D + M), jnp.float32)
    w2blk = w2blk.at[0:64, 0:D].set(W_r2a)
    w2blk = w2blk.at[64:128, D:2 * D].set(W_r2b)
    w2blk = w2blk.at[128:192, 2 * D:2 * D + M].set(W_er2)

    w1, w2, gate = pl.pallas_call(
        _edge_dense_body,
        grid=(E // BE,),
        in_specs=[
            pl.BlockSpec((BE, 16), lambda i: (i, 0)),
            pl.BlockSpec((BE, 3), lambda i: (i, 0)),
            _full((NB, 192)), _full((192, 2 * D + M)),
            _full((4, D)), _full((4, D)),
        ],
        out_specs=[
            pl.BlockSpec((BE, D), lambda i: (i, 0)),
            pl.BlockSpec((BE, D), lambda i: (i, 0)),
            pl.BlockSpec((BE, M), lambda i: (i, 0)),
        ],
        out_shape=[
            jax.ShapeDtypeStruct((E, D), jnp.float32),
            jax.ShapeDtypeStruct((E, D), jnp.float32),
            jax.ShapeDtypeStruct((E, M), jnp.float32),
        ],
    )(vec.reshape(E, 16), shifts, w1cat, w2blk, W_sh1, W_sh2)

    m1 = _interact_sc(h0, w1.reshape(NW, NCH, C, D), ei4)

    h1 = pl.pallas_call(
        _node1_body,
        out_shape=jax.ShapeDtypeStruct((N, D), jnp.float32),
    )(m1.reshape(NCORES, N_PAD, D), W_lin1)

    m2 = _interact_sc(h1, w2.reshape(NW, NCH, C, D), ei4)

    node_labels, eh = pl.pallas_call(
        _node2_body,
        out_shape=[
            jax.ShapeDtypeStruct((N, M), jnp.float32),
            jax.ShapeDtypeStruct((N, M), jnp.float32),
        ],
    )(m2.reshape(NCORES, N_PAD, D), h1, W_lin2,
      W_node[:D], W_node[D:], W_eh[:D], W_eh[D:])

    edge_labels = _readout_sc(eh, gate.reshape(NW, NCH, C, M), ei4)
    return node_labels, edge_labels.reshape(E, M)
